# Initial kernel scaffold; baseline (speedup 1.0000x reference)
import jax
import jax.numpy as jnp
from jax.experimental import pallas as pl


def kernel(input, rois):
    raise NotImplementedError("write your pallas kernel here")


# speedup vs baseline: 62.9287x; 62.9287x over previous
import functools

import jax
import jax.numpy as jnp
from jax import lax
from jax.experimental import pallas as pl
from jax.experimental.pallas import tpu as pltpu
from jax.experimental.pallas import tpu_sc as plsc

_SCALE = 0.125
_P = 7
_WIN = 22
_C = 128
_NV = _C // 16
_PP = _P * _P


def _make_params(rois, H, W):
    b = rois[:, 0].astype(jnp.int32)
    rsw = jnp.round(rois[:, 1] * _SCALE).astype(jnp.int32)
    rsh = jnp.round(rois[:, 2] * _SCALE).astype(jnp.int32)
    rew = jnp.round(rois[:, 3] * _SCALE).astype(jnp.int32)
    reh = jnp.round(rois[:, 4] * _SCALE).astype(jnp.int32)
    roi_w = jnp.maximum(rew - rsw + 1, 1)
    roi_h = jnp.maximum(reh - rsh + 1, 1)
    bin_h = roi_h.astype(jnp.float32) / _P
    bin_w = roi_w.astype(jnp.float32) / _P
    p = jnp.arange(_P)
    hstart = jnp.clip(jnp.floor(p * bin_h[:, None]).astype(jnp.int32) + rsh[:, None], 0, H)
    hend = jnp.clip(jnp.ceil((p + 1) * bin_h[:, None]).astype(jnp.int32) + rsh[:, None], 0, H)
    wstart = jnp.clip(jnp.floor(p * bin_w[:, None]).astype(jnp.int32) + rsw[:, None], 0, W)
    wend = jnp.clip(jnp.ceil((p + 1) * bin_w[:, None]).astype(jnp.int32) + rsw[:, None], 0, W)
    h0 = jnp.minimum(hstart[:, 0], H - _WIN)
    w0 = jnp.minimum(wstart[:, 0], W - _WIN)
    row0 = b * H + h0
    return jnp.concatenate(
        [row0[:, None], w0[:, None],
         hstart - h0[:, None], hend - h0[:, None],
         wstart - w0[:, None], wend - w0[:, None],
         jnp.zeros((rois.shape[0], 2), jnp.int32)], axis=1)


def _roi_body(x_hbm, params_hbm, out_hbm, params_v, win, outb, n_per_w, nc):
    wid = lax.axis_index("s") * nc + lax.axis_index("c")
    base = wid * n_per_w
    pltpu.sync_copy(params_hbm.at[pl.ds(base, n_per_w)], params_v)
    iota49 = lax.iota(jnp.int32, 16) * _PP
    neg = jnp.full((16,), -jnp.inf, jnp.float32)

    def one_roi(r, _):
        v0 = params_v[r, pl.ds(0, 16)]
        v1 = params_v[r, pl.ds(16, 16)]
        row0 = v0[0]
        w0 = v0[1]
        pltpu.sync_copy(x_hbm.at[pl.ds(row0, _WIN), pl.ds(w0, _WIN)], win)
        for py in range(_P):
            hs = v0[2 + py]
            he = v0[2 + _P + py]
            for px in range(_P):
                ws = v1[px]
                we = v1[_P + px]

                def h_body(h, accs):
                    def w_body(w, accs2):
                        return tuple(
                            jnp.maximum(accs2[v], win[h, w, pl.ds(16 * v, 16)])
                            for v in range(_NV))
                    return lax.fori_loop(ws, we, w_body, accs)

                accs = lax.fori_loop(hs, he, h_body, (neg,) * _NV)
                empty = (he <= hs) | (we <= ws)
                for v in range(_NV):
                    val = jnp.where(empty, jnp.float32(0.0), accs[v])
                    idx = iota49 + (v * 16 * _PP + py * _P + px)
                    plsc.store_scatter(outb, [idx], val)
        pltpu.sync_copy(outb, out_hbm.at[base + r])
        return _

    lax.fori_loop(0, n_per_w, one_roi, None)


def kernel(input, rois):
    N, C, H, W = input.shape
    nroi = rois.shape[0]
    x_hwc = jnp.transpose(input, (0, 2, 3, 1)).reshape(N * H, W, C)
    params = _make_params(rois, H, W)

    info = plsc.get_sparse_core_info()
    nc, ns = info.num_cores, info.num_subcores
    nw = nc * ns
    n_per_w = nroi // nw
    mesh = plsc.VectorSubcoreMesh(core_axis_name="c", subcore_axis_name="s")

    sc = pl.kernel(
        functools.partial(_roi_body, n_per_w=n_per_w, nc=nc),
        out_type=jax.ShapeDtypeStruct((nroi, _C * _PP), jnp.float32),
        mesh=mesh,
        scratch_types=[
            pltpu.VMEM((n_per_w, 32), jnp.int32),
            pltpu.VMEM((_WIN, _WIN, _C), jnp.float32),
            pltpu.VMEM((_C * _PP,), jnp.float32),
        ],
        compiler_params=pltpu.CompilerParams(use_tc_tiling_on_sc=False, needs_layout_passes=False),
        name="roipool_sc",
    )
    out = sc(x_hwc, params)
    return out.reshape(nroi, _C, _P, _P)


# speedup vs baseline: 84.2403x; 1.3387x over previous
import functools

import jax
import jax.numpy as jnp
from jax import lax
from jax.experimental import pallas as pl
from jax.experimental.pallas import tpu as pltpu
from jax.experimental.pallas import tpu_sc as plsc

_SCALE = 0.125
_P = 7
_WIN = 22
_C = 128
_NV = _C // 16
_PP = _P * _P
_ROW = _P * _C


def _make_params(rois, H, W):
    b = rois[:, 0].astype(jnp.int32)
    rsw = jnp.round(rois[:, 1] * _SCALE).astype(jnp.int32)
    rsh = jnp.round(rois[:, 2] * _SCALE).astype(jnp.int32)
    rew = jnp.round(rois[:, 3] * _SCALE).astype(jnp.int32)
    reh = jnp.round(rois[:, 4] * _SCALE).astype(jnp.int32)
    roi_w = jnp.maximum(rew - rsw + 1, 1)
    roi_h = jnp.maximum(reh - rsh + 1, 1)
    bin_h = roi_h.astype(jnp.float32) / _P
    bin_w = roi_w.astype(jnp.float32) / _P
    p = jnp.arange(_P)
    hstart = jnp.clip(jnp.floor(p * bin_h[:, None]).astype(jnp.int32) + rsh[:, None], 0, H)
    hend = jnp.clip(jnp.ceil((p + 1) * bin_h[:, None]).astype(jnp.int32) + rsh[:, None], 0, H)
    wstart = jnp.clip(jnp.floor(p * bin_w[:, None]).astype(jnp.int32) + rsw[:, None], 0, W)
    wend = jnp.clip(jnp.ceil((p + 1) * bin_w[:, None]).astype(jnp.int32) + rsw[:, None], 0, W)
    h0 = jnp.minimum(hstart[:, 0], H - _WIN)
    w0 = jnp.minimum(wstart[:, 0], W - _WIN)
    row0 = b * H + h0
    return jnp.concatenate(
        [row0[:, None], w0[:, None],
         hstart - h0[:, None], hend - h0[:, None],
         wstart - w0[:, None], wend - w0[:, None],
         jnp.zeros((rois.shape[0], 2), jnp.int32)], axis=1).astype(jnp.int16)


def _roi_body(x_hbm, params_hbm, out_hbm, pv, win, outb, semw0, semw1,
              semo0, semo1, n_per_w, nc):
    wid = lax.axis_index("s") * nc + lax.axis_index("c")
    base = wid * n_per_w
    pltpu.sync_copy(params_hbm.at[pl.ds(base, n_per_w)], pv)
    neg = jnp.full((16,), -jnp.inf, jnp.float32)

    def pget(vv32, i):
        s = vv32[i // 2]
        return (s & 0xFFFF) if i % 2 == 0 else lax.shift_right_logical(s, 16)

    def win_src(vv32):
        row0 = pget(vv32, 0)
        w0 = pget(vv32, 1)
        return x_hbm.at[pl.ds(row0, _WIN), pl.ds(w0, _WIN)]

    def pvec(r):
        return plsc.bitcast(pv[r, pl.ds(0, 32)], jnp.int32)

    pltpu.async_copy(win_src(pvec(0)), win.at[0], semw0)

    def one_roi(r, _):
        nb = (r + 1) & 1
        cb = r & 1

        @pl.when(r + 1 < n_per_w)
        def _fire():
            src = win_src(pvec(r + 1))

            @pl.when(nb == 0)
            def _():
                pltpu.async_copy(src, win.at[0], semw0)

            @pl.when(nb == 1)
            def _():
                pltpu.async_copy(src, win.at[1], semw1)

        dummy = x_hbm.at[pl.ds(0, _WIN), pl.ds(0, _WIN)]

        @pl.when(cb == 0)
        def _():
            pltpu.make_async_copy(dummy, win.at[0], semw0).wait()

        @pl.when(cb == 1)
        def _():
            pltpu.make_async_copy(dummy, win.at[1], semw1).wait()

        vv32 = pvec(r)
        out_off = (base + r) * (_PP * _C)
        for py in range(_P):
            ob = py & 1
            osem = semo0 if ob == 0 else semo1
            obuf = outb.at[ob]
            if py >= 2:
                pltpu.make_async_copy(obuf, out_hbm.at[pl.ds(0, _ROW)],
                                      osem).wait()
            else:
                @pl.when(r > 0)
                def _():
                    pltpu.make_async_copy(obuf, out_hbm.at[pl.ds(0, _ROW)],
                                          osem).wait()
            hs = pget(vv32, 2 + py)
            he = pget(vv32, 2 + _P + py)
            for px in range(_P):
                ws = pget(vv32, 2 + 2 * _P + px)
                we = pget(vv32, 2 + 3 * _P + px)

                def h_body(h, accs):
                    def w_body(w, accs2):
                        return tuple(
                            jnp.maximum(accs2[v], win[cb, h, w, pl.ds(16 * v, 16)])
                            for v in range(_NV))
                    return lax.fori_loop(ws, we, w_body, accs)

                accs = lax.fori_loop(hs, he, h_body, (neg,) * _NV)
                empty = (he <= hs) | (we <= ws)
                for v in range(_NV):
                    outb[ob, pl.ds(px * _C + 16 * v, 16)] = jnp.where(
                        empty, jnp.float32(0.0), accs[v])
            pltpu.async_copy(
                obuf, out_hbm.at[pl.ds(out_off + py * _ROW, _ROW)], osem)

    lax.fori_loop(0, n_per_w, one_roi, None)
    pltpu.make_async_copy(outb.at[0], out_hbm.at[pl.ds(0, _ROW)], semo1).wait()
    pltpu.make_async_copy(outb.at[1], out_hbm.at[pl.ds(0, _ROW)], semo0).wait()


def kernel(input, rois):
    N, C, H, W = input.shape
    nroi = rois.shape[0]
    x_hwc = jnp.transpose(input, (0, 2, 3, 1)).reshape(N * H, W, C)
    params = _make_params(rois, H, W)

    info = plsc.get_sparse_core_info()
    nc, ns = info.num_cores, info.num_subcores
    nw = nc * ns
    n_per_w = nroi // nw
    mesh = plsc.VectorSubcoreMesh(core_axis_name="c", subcore_axis_name="s")

    sc = pl.kernel(
        functools.partial(_roi_body, n_per_w=n_per_w, nc=nc),
        out_type=jax.ShapeDtypeStruct((nroi * _PP * _C,), jnp.float32),
        mesh=mesh,
        scratch_types=[
            pltpu.VMEM((n_per_w, 32), jnp.int16),
            pltpu.VMEM((2, _WIN, _WIN, _C), jnp.float32),
            pltpu.VMEM((2, _P * _C), jnp.float32),
            pltpu.SemaphoreType.DMA,
            pltpu.SemaphoreType.DMA,
            pltpu.SemaphoreType.DMA,
            pltpu.SemaphoreType.DMA,
        ],
        compiler_params=pltpu.CompilerParams(
            use_tc_tiling_on_sc=False, needs_layout_passes=False),
        name="roipool_sc",
    )
    out = sc(x_hwc, params)
    return out.reshape(nroi, _PP, _C).transpose(0, 2, 1).reshape(
        nroi, _C, _P, _P)


# speedup vs baseline: 131.7978x; 1.5645x over previous
import functools

import jax
import jax.numpy as jnp
from jax import lax
from jax.experimental import pallas as pl
from jax.experimental.pallas import tpu as pltpu
from jax.experimental.pallas import tpu_sc as plsc

_SCALE = 0.125
_P = 7
_WIN = 22
_C = 128
_NV = _C // 16
_PP = _P * _P
_ROW = _P * _C


def _make_params(rois, H, W):
    b = rois[:, 0].astype(jnp.int32)
    rsw = jnp.round(rois[:, 1] * _SCALE).astype(jnp.int32)
    rsh = jnp.round(rois[:, 2] * _SCALE).astype(jnp.int32)
    rew = jnp.round(rois[:, 3] * _SCALE).astype(jnp.int32)
    reh = jnp.round(rois[:, 4] * _SCALE).astype(jnp.int32)
    roi_w = jnp.maximum(rew - rsw + 1, 1)
    roi_h = jnp.maximum(reh - rsh + 1, 1)
    bin_h = roi_h.astype(jnp.float32) / _P
    bin_w = roi_w.astype(jnp.float32) / _P
    p = jnp.arange(_P)
    hstart = jnp.clip(jnp.floor(p * bin_h[:, None]).astype(jnp.int32) + rsh[:, None], 0, H)
    hend = jnp.clip(jnp.ceil((p + 1) * bin_h[:, None]).astype(jnp.int32) + rsh[:, None], 0, H)
    wstart = jnp.clip(jnp.floor(p * bin_w[:, None]).astype(jnp.int32) + rsw[:, None], 0, W)
    wend = jnp.clip(jnp.ceil((p + 1) * bin_w[:, None]).astype(jnp.int32) + rsw[:, None], 0, W)
    h0 = jnp.minimum(hstart[:, 0], H - _WIN)
    w0 = jnp.minimum(wstart[:, 0], W - _WIN)
    row0 = b * H + h0
    return jnp.concatenate(
        [row0[:, None], w0[:, None],
         hstart - h0[:, None], hend - h0[:, None],
         wstart - w0[:, None], wend - w0[:, None],
         jnp.zeros((rois.shape[0], 2), jnp.int32)], axis=1).astype(jnp.int16)


def _roi_body(x_hbm, params_hbm, out_hbm, pv, win, outb, wb, semw0, semw1,
              semo0, semo1, n_per_w, nc):
    wid = lax.axis_index("s") * nc + lax.axis_index("c")
    base = wid * n_per_w
    pltpu.sync_copy(params_hbm.at[pl.ds(base, n_per_w)], pv)
    neg = jnp.full((16,), -jnp.inf, jnp.float32)

    def pget(vv32, i):
        s = vv32[i // 2]
        return (s & 0xFFFF) if i % 2 == 0 else lax.shift_right_logical(s, 16)

    def win_src(vv32):
        row0 = pget(vv32, 0)
        w0 = pget(vv32, 1)
        return x_hbm.at[pl.ds(row0, _WIN), pl.ds(w0, _WIN)]

    def pvec(r):
        return plsc.bitcast(pv[r, pl.ds(0, 32)], jnp.int32)

    pltpu.async_copy(win_src(pvec(0)), win.at[0], semw0)

    def one_roi(r, _):
        nb = (r + 1) & 1
        cb = r & 1

        @pl.when(r + 1 < n_per_w)
        def _fire():
            src = win_src(pvec(r + 1))

            @pl.when(nb == 0)
            def _():
                pltpu.async_copy(src, win.at[0], semw0)

            @pl.when(nb == 1)
            def _():
                pltpu.async_copy(src, win.at[1], semw1)

        dummy = x_hbm.at[pl.ds(0, _WIN), pl.ds(0, _WIN)]

        @pl.when(cb == 0)
        def _():
            pltpu.make_async_copy(dummy, win.at[0], semw0).wait()

        @pl.when(cb == 1)
        def _():
            pltpu.make_async_copy(dummy, win.at[1], semw1).wait()

        vv32 = pvec(r)
        hs_l = [pget(vv32, 2 + py) for py in range(_P)]
        he_l = [pget(vv32, 2 + _P + py) for py in range(_P)]
        for px in range(_P):
            wb[px] = pget(vv32, 2 + 2 * _P + px)
            wb[_P + px] = pget(vv32, 2 + 3 * _P + px)
        out_off = (base + r) * (_PP * _C)
        for py in range(_P):
            ob = py & 1
            osem = semo0 if ob == 0 else semo1
            obuf = outb.at[ob]
            if py >= 2:
                pltpu.make_async_copy(obuf, out_hbm.at[pl.ds(0, _ROW)],
                                      osem).wait()
            else:
                @pl.when(r > 0)
                def _():
                    pltpu.make_async_copy(obuf, out_hbm.at[pl.ds(0, _ROW)],
                                          osem).wait()
            hs, he = hs_l[py], he_l[py]
            h1 = he - 1
            h0c = jnp.minimum(hs, _WIN - 1)
            h1c = jnp.maximum(h1, 0)

            def px_body(px, _2):
                ws = wb[px]
                we = wb[_P + px]
                w1 = we - 1
                w0c = jnp.minimum(ws, _WIN - 1)
                w1c = jnp.maximum(w1, 0)

                def pix(h, w, v):
                    return win[cb, h, w, pl.ds(16 * v, 16)]

                accs = tuple(
                    jnp.maximum(
                        jnp.maximum(pix(h0c, w0c, v), pix(h0c, w1c, v)),
                        jnp.maximum(pix(h1c, w0c, v), pix(h1c, w1c, v)))
                    for v in range(_NV))

                def ec_body(w, a):
                    return tuple(
                        jnp.maximum(a[v], jnp.maximum(pix(h0c, w, v),
                                                      pix(h1c, w, v)))
                        for v in range(_NV))
                accs = lax.fori_loop(ws + 1, w1, ec_body, accs)

                def mr_body(h, a):
                    def w_body(w, a2):
                        return tuple(
                            jnp.maximum(a2[v], pix(h, w, v))
                            for v in range(_NV))
                    return lax.fori_loop(ws, we, w_body, a)
                accs = lax.fori_loop(hs + 1, h1, mr_body, accs)

                empty = (he <= hs) | (we <= ws)
                for v in range(_NV):
                    outb[ob, pl.ds(px * _C + 16 * v, 16)] = jnp.where(
                        empty, jnp.float32(0.0), accs[v])

            lax.fori_loop(0, _P, px_body, None)
            pltpu.async_copy(
                obuf, out_hbm.at[pl.ds(out_off + py * _ROW, _ROW)], osem)

    lax.fori_loop(0, n_per_w, one_roi, None)
    pltpu.make_async_copy(outb.at[0], out_hbm.at[pl.ds(0, _ROW)], semo1).wait()
    pltpu.make_async_copy(outb.at[1], out_hbm.at[pl.ds(0, _ROW)], semo0).wait()


def kernel(input, rois):
    N, C, H, W = input.shape
    nroi = rois.shape[0]
    x_hwc = jnp.transpose(input, (0, 2, 3, 1)).reshape(N * H, W, C)
    params = _make_params(rois, H, W)

    info = plsc.get_sparse_core_info()
    nc, ns = info.num_cores, info.num_subcores
    nw = nc * ns
    n_per_w = nroi // nw
    mesh = plsc.VectorSubcoreMesh(core_axis_name="c", subcore_axis_name="s")

    sc = pl.kernel(
        functools.partial(_roi_body, n_per_w=n_per_w, nc=nc),
        out_type=jax.ShapeDtypeStruct((nroi * _PP * _C,), jnp.float32),
        mesh=mesh,
        scratch_types=[
            pltpu.VMEM((n_per_w, 32), jnp.int16),
            pltpu.VMEM((2, _WIN, _WIN, _C), jnp.float32),
            pltpu.VMEM((2, _P * _C), jnp.float32),
            pltpu.SMEM((2 * _P,), jnp.int32),
            pltpu.SemaphoreType.DMA,
            pltpu.SemaphoreType.DMA,
            pltpu.SemaphoreType.DMA,
            pltpu.SemaphoreType.DMA,
        ],
        compiler_params=pltpu.CompilerParams(
            use_tc_tiling_on_sc=False, needs_layout_passes=False),
        name="roipool_sc",
    )
    out = sc(x_hwc, params)
    return out.reshape(nroi, _PP, _C).transpose(0, 2, 1).reshape(
        nroi, _C, _P, _P)


# speedup vs baseline: 133.9583x; 1.0164x over previous
import functools

import jax
import jax.numpy as jnp
from jax import lax
from jax.experimental import pallas as pl
from jax.experimental.pallas import tpu as pltpu
from jax.experimental.pallas import tpu_sc as plsc

_SCALE = 0.125
_P = 7
_WIN = 22
_C = 128
_NV = _C // 16
_PP = _P * _P
_ROW = _P * _C


def _make_params(rois, H, W):
    b = rois[:, 0].astype(jnp.int32)
    rsw = jnp.round(rois[:, 1] * _SCALE).astype(jnp.int32)
    rsh = jnp.round(rois[:, 2] * _SCALE).astype(jnp.int32)
    rew = jnp.round(rois[:, 3] * _SCALE).astype(jnp.int32)
    reh = jnp.round(rois[:, 4] * _SCALE).astype(jnp.int32)
    roi_w = jnp.maximum(rew - rsw + 1, 1)
    roi_h = jnp.maximum(reh - rsh + 1, 1)
    bin_h = roi_h.astype(jnp.float32) / _P
    bin_w = roi_w.astype(jnp.float32) / _P
    p = jnp.arange(_P)
    hstart = jnp.clip(jnp.floor(p * bin_h[:, None]).astype(jnp.int32) + rsh[:, None], 0, H)
    hend = jnp.clip(jnp.ceil((p + 1) * bin_h[:, None]).astype(jnp.int32) + rsh[:, None], 0, H)
    wstart = jnp.clip(jnp.floor(p * bin_w[:, None]).astype(jnp.int32) + rsw[:, None], 0, W)
    wend = jnp.clip(jnp.ceil((p + 1) * bin_w[:, None]).astype(jnp.int32) + rsw[:, None], 0, W)
    h0 = jnp.minimum(hstart[:, 0], H - _WIN)
    w0 = jnp.minimum(wstart[:, 0], W - _WIN)
    row0 = b * H + h0
    return jnp.concatenate(
        [row0[:, None], w0[:, None],
         hstart - h0[:, None], hend - h0[:, None],
         wstart - w0[:, None], wend - w0[:, None],
         jnp.zeros((rois.shape[0], 2), jnp.int32)], axis=1).astype(jnp.int16)


def _roi_body(x_hbm, params_hbm, out_hbm, pv, win, outb, wb, semw0, semw1,
              semo0, semo1, n_per_w, nc):
    wid = lax.axis_index("s") * nc + lax.axis_index("c")
    base = wid * n_per_w
    pltpu.sync_copy(params_hbm.at[pl.ds(base, n_per_w)], pv)
    neg = jnp.full((16,), -jnp.inf, jnp.float32)

    def pget(vv32, i):
        s = vv32[i // 2]
        return (s & 0xFFFF) if i % 2 == 0 else lax.shift_right_logical(s, 16)

    def pvec(r):
        return plsc.bitcast(pv[r, pl.ds(0, 32)], jnp.int32)

    def fire(r, buf, sem):
        vv32 = pvec(r)
        row0 = pget(vv32, 0)
        w0 = pget(vv32, 1)
        rows = pget(vv32, 2 + 2 * _P - 1)

        def fb(i, _):
            pltpu.async_copy(
                x_hbm.at[pl.ds(row0 + i, 1), pl.ds(w0, _WIN)],
                win.at[buf, pl.ds(i, 1)], sem)
        lax.fori_loop(0, rows, fb, None)

    def drain(r, buf, sem):
        rows = pget(pvec(r), 2 + 2 * _P - 1)

        def db(i, _):
            pltpu.make_async_copy(
                x_hbm.at[pl.ds(0, 1), pl.ds(0, _WIN)],
                win.at[buf, pl.ds(0, 1)], sem).wait()
        lax.fori_loop(0, rows, db, None)

    fire(0, 0, semw0)

    def one_roi(r, _):
        nb = (r + 1) & 1
        cb = r & 1

        @pl.when(r + 1 < n_per_w)
        def _fire():
            @pl.when(nb == 0)
            def _():
                fire(r + 1, 0, semw0)

            @pl.when(nb == 1)
            def _():
                fire(r + 1, 1, semw1)

        @pl.when(cb == 0)
        def _():
            drain(r, 0, semw0)

        @pl.when(cb == 1)
        def _():
            drain(r, 1, semw1)

        vv32 = pvec(r)
        hs_l = [pget(vv32, 2 + py) for py in range(_P)]
        he_l = [pget(vv32, 2 + _P + py) for py in range(_P)]
        for px in range(_P):
            wb[px] = pget(vv32, 2 + 2 * _P + px)
            wb[_P + px] = pget(vv32, 2 + 3 * _P + px)
        out_off = (base + r) * (_PP * _C)
        for py in range(_P):
            ob = py & 1
            osem = semo0 if ob == 0 else semo1
            obuf = outb.at[ob]
            if py >= 2:
                pltpu.make_async_copy(obuf, out_hbm.at[pl.ds(0, _ROW)],
                                      osem).wait()
            else:
                @pl.when(r > 0)
                def _():
                    pltpu.make_async_copy(obuf, out_hbm.at[pl.ds(0, _ROW)],
                                          osem).wait()
            hs, he = hs_l[py], he_l[py]
            h1 = he - 1
            h0c = jnp.minimum(hs, _WIN - 1)
            h1c = jnp.maximum(h1, 0)

            def px_body(px, _2):
                ws = wb[px]
                we = wb[_P + px]
                w1 = we - 1
                w0c = jnp.minimum(ws, _WIN - 1)
                w1c = jnp.maximum(w1, 0)

                def pix(h, w, v):
                    return win[cb, h, w, pl.ds(16 * v, 16)]

                accs = tuple(
                    jnp.maximum(
                        jnp.maximum(pix(h0c, w0c, v), pix(h0c, w1c, v)),
                        jnp.maximum(pix(h1c, w0c, v), pix(h1c, w1c, v)))
                    for v in range(_NV))

                def ec_body(w, a):
                    return tuple(
                        jnp.maximum(a[v], jnp.maximum(pix(h0c, w, v),
                                                      pix(h1c, w, v)))
                        for v in range(_NV))
                accs = lax.fori_loop(ws + 1, w1, ec_body, accs)

                def mr_body(h, a):
                    def w_body(w, a2):
                        return tuple(
                            jnp.maximum(a2[v], pix(h, w, v))
                            for v in range(_NV))
                    return lax.fori_loop(ws, we, w_body, a)
                accs = lax.fori_loop(hs + 1, h1, mr_body, accs)

                empty = (he <= hs) | (we <= ws)
                for v in range(_NV):
                    outb[ob, pl.ds(px * _C + 16 * v, 16)] = jnp.where(
                        empty, jnp.float32(0.0), accs[v])

            lax.fori_loop(0, _P, px_body, None)
            pltpu.async_copy(
                obuf, out_hbm.at[pl.ds(out_off + py * _ROW, _ROW)], osem)

    lax.fori_loop(0, n_per_w, one_roi, None)
    pltpu.make_async_copy(outb.at[0], out_hbm.at[pl.ds(0, _ROW)], semo1).wait()
    pltpu.make_async_copy(outb.at[1], out_hbm.at[pl.ds(0, _ROW)], semo0).wait()


def kernel(input, rois):
    N, C, H, W = input.shape
    nroi = rois.shape[0]
    x_hwc = jnp.transpose(input, (0, 2, 3, 1)).reshape(N * H, W, C)
    params = _make_params(rois, H, W)

    info = plsc.get_sparse_core_info()
    nc, ns = info.num_cores, info.num_subcores
    nw = nc * ns
    n_per_w = nroi // nw
    mesh = plsc.VectorSubcoreMesh(core_axis_name="c", subcore_axis_name="s")

    sc = pl.kernel(
        functools.partial(_roi_body, n_per_w=n_per_w, nc=nc),
        out_type=jax.ShapeDtypeStruct((nroi * _PP * _C,), jnp.float32),
        mesh=mesh,
        scratch_types=[
            pltpu.VMEM((n_per_w, 32), jnp.int16),
            pltpu.VMEM((2, _WIN, _WIN, _C), jnp.float32),
            pltpu.VMEM((2, _P * _C), jnp.float32),
            pltpu.SMEM((2 * _P,), jnp.int32),
            pltpu.SemaphoreType.DMA,
            pltpu.SemaphoreType.DMA,
            pltpu.SemaphoreType.DMA,
            pltpu.SemaphoreType.DMA,
        ],
        compiler_params=pltpu.CompilerParams(
            use_tc_tiling_on_sc=False, needs_layout_passes=False),
        name="roipool_sc",
    )
    out = sc(x_hwc, params)
    return out.reshape(nroi, _PP, _C).transpose(0, 2, 1).reshape(
        nroi, _C, _P, _P)


# speedup vs baseline: 138.8735x; 1.0367x over previous
import functools

import jax
import jax.numpy as jnp
from jax import lax
from jax.experimental import pallas as pl
from jax.experimental.pallas import tpu as pltpu
from jax.experimental.pallas import tpu_sc as plsc

_SCALE = 0.125
_P = 7
_WIN = 22
_C = 128
_NV = _C // 16
_PP = _P * _P
_ROW = _P * _C


def _make_params(rois, H, W):
    b = rois[:, 0].astype(jnp.int32)
    rsw = jnp.round(rois[:, 1] * _SCALE).astype(jnp.int32)
    rsh = jnp.round(rois[:, 2] * _SCALE).astype(jnp.int32)
    rew = jnp.round(rois[:, 3] * _SCALE).astype(jnp.int32)
    reh = jnp.round(rois[:, 4] * _SCALE).astype(jnp.int32)
    roi_w = jnp.maximum(rew - rsw + 1, 1)
    roi_h = jnp.maximum(reh - rsh + 1, 1)
    bin_h = roi_h.astype(jnp.float32) / _P
    bin_w = roi_w.astype(jnp.float32) / _P
    p = jnp.arange(_P)
    hstart = jnp.clip(jnp.floor(p * bin_h[:, None]).astype(jnp.int32) + rsh[:, None], 0, H)
    hend = jnp.clip(jnp.ceil((p + 1) * bin_h[:, None]).astype(jnp.int32) + rsh[:, None], 0, H)
    wstart = jnp.clip(jnp.floor(p * bin_w[:, None]).astype(jnp.int32) + rsw[:, None], 0, W)
    wend = jnp.clip(jnp.ceil((p + 1) * bin_w[:, None]).astype(jnp.int32) + rsw[:, None], 0, W)
    h0 = jnp.minimum(hstart[:, 0], H - _WIN)
    w0 = jnp.minimum(wstart[:, 0], W - _WIN)
    row0 = b * H + h0
    return jnp.concatenate(
        [row0[:, None], w0[:, None],
         hstart - h0[:, None], hend - h0[:, None],
         wstart - w0[:, None], wend - w0[:, None],
         jnp.zeros((rois.shape[0], 2), jnp.int32)], axis=1).astype(jnp.int16)


def _roi_body(x_hbm, params_hbm, out_hbm, pv, win, outb, wb, semw0, semw1,
              semo0, semo1, n_per_w, nc):
    wid = lax.axis_index("s") * nc + lax.axis_index("c")
    base = wid * n_per_w
    pltpu.sync_copy(params_hbm.at[pl.ds(base, n_per_w)], pv)
    neg = jnp.full((16,), -jnp.inf, jnp.float32)

    def pget(vv32, i):
        s = vv32[i // 2]
        return (s & 0xFFFF) if i % 2 == 0 else lax.shift_right_logical(s, 16)

    def pvec(r):
        return plsc.bitcast(pv[r, pl.ds(0, 32)], jnp.int32)

    def fire(r, buf, sem):
        vv32 = pvec(r)
        row0 = pget(vv32, 0)
        w0 = pget(vv32, 1)
        rows = pget(vv32, 2 + 2 * _P - 1)

        def fb(i, _):
            pltpu.async_copy(
                x_hbm.at[pl.ds(row0 + i, 1), pl.ds(w0, _WIN)],
                win.at[buf, pl.ds(i, 1)], sem)
        lax.fori_loop(0, rows, fb, None)

    def drain(r, buf, sem):
        rows = pget(pvec(r), 2 + 2 * _P - 1)

        def db(i, _):
            pltpu.make_async_copy(
                x_hbm.at[pl.ds(0, 1), pl.ds(0, _WIN)],
                win.at[buf, pl.ds(0, 1)], sem).wait()
        lax.fori_loop(0, rows, db, None)

    fire(0, 0, semw0)

    def one_roi(r, _):
        nb = (r + 1) & 1
        cb = r & 1

        @pl.when(r + 1 < n_per_w)
        def _fire():
            @pl.when(nb == 0)
            def _():
                fire(r + 1, 0, semw0)

            @pl.when(nb == 1)
            def _():
                fire(r + 1, 1, semw1)

        @pl.when(cb == 0)
        def _():
            drain(r, 0, semw0)

        @pl.when(cb == 1)
        def _():
            drain(r, 1, semw1)

        vv32 = pvec(r)
        hs_l = [pget(vv32, 2 + py) for py in range(_P)]
        he_l = [pget(vv32, 2 + _P + py) for py in range(_P)]
        for px in range(_P):
            wb[px] = pget(vv32, 2 + 2 * _P + px)
            wb[_P + px] = pget(vv32, 2 + 3 * _P + px)
        out_off = (base + r) * (_PP * _C)
        for py in range(_P):
            ob = py & 1
            osem = semo0 if ob == 0 else semo1
            obuf = outb.at[ob]
            if py >= 2:
                pltpu.make_async_copy(obuf, out_hbm.at[pl.ds(0, _ROW)],
                                      osem).wait()
            else:
                @pl.when(r > 0)
                def _():
                    pltpu.make_async_copy(obuf, out_hbm.at[pl.ds(0, _ROW)],
                                          osem).wait()
            hs, he = hs_l[py], he_l[py]
            h1 = he - 1
            h0c = jnp.minimum(hs, _WIN - 1)
            h1c = jnp.maximum(h1, 0)

            def px_body(px, _2):
                ws = wb[px]
                we = wb[_P + px]
                w1 = we - 1
                w0c = jnp.minimum(ws, _WIN - 1)
                w1c = jnp.maximum(w1, 0)

                def pix(h, w, v):
                    return win[cb, h, w, pl.ds(16 * v, 16)]

                accs = tuple(
                    jnp.maximum(
                        jnp.maximum(pix(h0c, w0c, v), pix(h0c, w1c, v)),
                        jnp.maximum(pix(h1c, w0c, v), pix(h1c, w1c, v)))
                    for v in range(_NV))

                empty = (he <= hs) | (we <= ws)
                big = ((w1 > ws + 1) | (h1 > hs + 1)) & jnp.logical_not(empty)

                @pl.when(jnp.logical_not(big))
                def _():
                    for v in range(_NV):
                        outb[ob, pl.ds(px * _C + 16 * v, 16)] = jnp.where(
                            empty, jnp.float32(0.0), accs[v])

                @pl.when(big)
                def _():
                    def ec_body(w, a):
                        return tuple(
                            jnp.maximum(a[v], jnp.maximum(pix(h0c, w, v),
                                                          pix(h1c, w, v)))
                            for v in range(_NV))
                    a2 = lax.fori_loop(ws + 1, w1, ec_body, accs)

                    def mr_body(h, a):
                        def w_body(w, a3):
                            return tuple(
                                jnp.maximum(a3[v], pix(h, w, v))
                                for v in range(_NV))
                        return lax.fori_loop(ws, we, w_body, a)
                    a2 = lax.fori_loop(hs + 1, h1, mr_body, a2)
                    for v in range(_NV):
                        outb[ob, pl.ds(px * _C + 16 * v, 16)] = a2[v]

            lax.fori_loop(0, _P, px_body, None)
            pltpu.async_copy(
                obuf, out_hbm.at[pl.ds(out_off + py * _ROW, _ROW)], osem)

    lax.fori_loop(0, n_per_w, one_roi, None)
    pltpu.make_async_copy(outb.at[0], out_hbm.at[pl.ds(0, _ROW)], semo1).wait()
    pltpu.make_async_copy(outb.at[1], out_hbm.at[pl.ds(0, _ROW)], semo0).wait()


def kernel(input, rois):
    N, C, H, W = input.shape
    nroi = rois.shape[0]
    x_hwc = jnp.transpose(input, (0, 2, 3, 1)).reshape(N * H, W, C)
    params = _make_params(rois, H, W)

    info = plsc.get_sparse_core_info()
    nc, ns = info.num_cores, info.num_subcores
    nw = nc * ns
    n_per_w = nroi // nw
    mesh = plsc.VectorSubcoreMesh(core_axis_name="c", subcore_axis_name="s")

    sc = pl.kernel(
        functools.partial(_roi_body, n_per_w=n_per_w, nc=nc),
        out_type=jax.ShapeDtypeStruct((nroi * _PP * _C,), jnp.float32),
        mesh=mesh,
        scratch_types=[
            pltpu.VMEM((n_per_w, 32), jnp.int16),
            pltpu.VMEM((2, _WIN, _WIN, _C), jnp.float32),
            pltpu.VMEM((2, _P * _C), jnp.float32),
            pltpu.SMEM((2 * _P,), jnp.int32),
            pltpu.SemaphoreType.DMA,
            pltpu.SemaphoreType.DMA,
            pltpu.SemaphoreType.DMA,
            pltpu.SemaphoreType.DMA,
        ],
        compiler_params=pltpu.CompilerParams(
            use_tc_tiling_on_sc=False, needs_layout_passes=False),
        name="roipool_sc",
    )
    out = sc(x_hwc, params)
    return out.reshape(nroi, _PP, _C).transpose(0, 2, 1).reshape(
        nroi, _C, _P, _P)


# speedup vs baseline: 139.1207x; 1.0018x over previous
import functools

import jax
import jax.numpy as jnp
from jax import lax
from jax.experimental import pallas as pl
from jax.experimental.pallas import tpu as pltpu
from jax.experimental.pallas import tpu_sc as plsc

_SCALE = 0.125
_P = 7
_WIN = 22
_C = 128
_NV = _C // 16
_PP = _P * _P
_ROW = _P * _C


def _make_params(rois, H, W):
    b = rois[:, 0].astype(jnp.int32)
    rsw = jnp.round(rois[:, 1] * _SCALE).astype(jnp.int32)
    rsh = jnp.round(rois[:, 2] * _SCALE).astype(jnp.int32)
    rew = jnp.round(rois[:, 3] * _SCALE).astype(jnp.int32)
    reh = jnp.round(rois[:, 4] * _SCALE).astype(jnp.int32)
    roi_w = jnp.maximum(rew - rsw + 1, 1)
    roi_h = jnp.maximum(reh - rsh + 1, 1)
    bin_h = roi_h.astype(jnp.float32) / _P
    bin_w = roi_w.astype(jnp.float32) / _P
    p = jnp.arange(_P)
    hstart = jnp.clip(jnp.floor(p * bin_h[:, None]).astype(jnp.int32) + rsh[:, None], 0, H)
    hend = jnp.clip(jnp.ceil((p + 1) * bin_h[:, None]).astype(jnp.int32) + rsh[:, None], 0, H)
    wstart = jnp.clip(jnp.floor(p * bin_w[:, None]).astype(jnp.int32) + rsw[:, None], 0, W)
    wend = jnp.clip(jnp.ceil((p + 1) * bin_w[:, None]).astype(jnp.int32) + rsw[:, None], 0, W)
    h0 = jnp.minimum(hstart[:, 0], H - _WIN)
    w0 = jnp.minimum(wstart[:, 0], W - _WIN)
    row0 = b * H + h0
    return jnp.concatenate(
        [row0[:, None], w0[:, None],
         hstart - h0[:, None], hend - h0[:, None],
         wstart - w0[:, None], wend - w0[:, None],
         jnp.zeros((rois.shape[0], 2), jnp.int32)], axis=1).astype(jnp.int16)


def _roi_body(x_hbm, params_hbm, out_hbm, pv, win, outb, wb, semw0, semw1,
              semo0, semo1, n_per_w, nc):
    wid = lax.axis_index("s") * nc + lax.axis_index("c")
    base = wid * n_per_w
    pltpu.sync_copy(params_hbm.at[pl.ds(base, n_per_w)], pv)
    neg = jnp.full((16,), -jnp.inf, jnp.float32)

    def pget(vv32, i):
        s = vv32[i // 2]
        return (s & 0xFFFF) if i % 2 == 0 else lax.shift_right_logical(s, 16)

    def pvec(r):
        return plsc.bitcast(pv[r, pl.ds(0, 32)], jnp.int32)

    def fire(r, buf, sem):
        vv32 = pvec(r)
        row0 = pget(vv32, 0)
        w0 = pget(vv32, 1)
        rows = pget(vv32, 2 + 2 * _P - 1)

        def fb(i, _):
            pltpu.async_copy(
                x_hbm.at[pl.ds(row0 + i, 1), pl.ds(w0, _WIN)],
                win.at[buf, pl.ds(i, 1)], sem)
        lax.fori_loop(0, rows, fb, None)

    def drain(r, buf, sem):
        rows = pget(pvec(r), 2 + 2 * _P - 1)

        def db(i, _):
            pltpu.make_async_copy(
                x_hbm.at[pl.ds(0, 1), pl.ds(0, _WIN)],
                win.at[buf, pl.ds(0, 1)], sem).wait()
        lax.fori_loop(0, rows, db, None)

    fire(0, 0, semw0)

    def one_roi(r, _):
        nb = (r + 1) & 1
        cb = r & 1

        @pl.when(r + 1 < n_per_w)
        def _fire():
            @pl.when(nb == 0)
            def _():
                fire(r + 1, 0, semw0)

            @pl.when(nb == 1)
            def _():
                fire(r + 1, 1, semw1)

        @pl.when(cb == 0)
        def _():
            drain(r, 0, semw0)

        @pl.when(cb == 1)
        def _():
            drain(r, 1, semw1)

        vv32 = pvec(r)
        hs_l = [pget(vv32, 2 + py) for py in range(_P)]
        he_l = [pget(vv32, 2 + _P + py) for py in range(_P)]
        for px in range(_P):
            wb[px] = pget(vv32, 2 + 2 * _P + px)
            wb[_P + px] = pget(vv32, 2 + 3 * _P + px)
        out_off = (base + r) * (_PP * _C)
        for py in range(_P):
            ob = py & 1
            osem = semo0 if ob == 0 else semo1
            obuf = outb.at[ob]
            if py >= 2:
                pltpu.make_async_copy(obuf, out_hbm.at[pl.ds(0, _ROW)],
                                      osem).wait()
            else:
                @pl.when(r > 0)
                def _():
                    pltpu.make_async_copy(obuf, out_hbm.at[pl.ds(0, _ROW)],
                                          osem).wait()
            hs, he = hs_l[py], he_l[py]
            h1 = he - 1
            h0c = jnp.minimum(hs, _WIN - 1)
            h1c = jnp.maximum(h1, 0)

            def px_body(px, _2):
                ws = wb[px]
                we = wb[_P + px]
                w1 = we - 1
                w0c = jnp.minimum(ws, _WIN - 1)
                w1c = jnp.maximum(w1, 0)

                def pix(h, w, v):
                    return win[cb, h, w, pl.ds(16 * v, 16)]

                accs = tuple(
                    jnp.maximum(
                        jnp.maximum(pix(h0c, w0c, v), pix(h0c, w1c, v)),
                        jnp.maximum(pix(h1c, w0c, v), pix(h1c, w1c, v)))
                    for v in range(_NV))

                empty = (he <= hs) | (we <= ws)
                big = ((w1 > ws + 1) | (h1 > hs + 1)) & jnp.logical_not(empty)

                @pl.when(jnp.logical_not(big))
                def _():
                    for v in range(_NV):
                        outb[ob, pl.ds(px * _C + 16 * v, 16)] = jnp.where(
                            empty, jnp.float32(0.0), accs[v])

                @pl.when(big)
                def _():
                    def ec_body(w, a):
                        return tuple(
                            jnp.maximum(a[v], jnp.maximum(pix(h0c, w, v),
                                                          pix(h1c, w, v)))
                            for v in range(_NV))
                    a2 = lax.fori_loop(ws + 1, w1, ec_body, accs)

                    def mr_body(h, a):
                        def w_body(w, a3):
                            return tuple(
                                jnp.maximum(a3[v], pix(h, w, v))
                                for v in range(_NV))
                        return lax.fori_loop(ws, we, w_body, a)
                    a2 = lax.fori_loop(hs + 1, h1, mr_body, a2)
                    for v in range(_NV):
                        outb[ob, pl.ds(px * _C + 16 * v, 16)] = a2[v]

            plsc.parallel_loop(0, _P)(lambda px: px_body(px, None))
            pltpu.async_copy(
                obuf, out_hbm.at[pl.ds(out_off + py * _ROW, _ROW)], osem)

    lax.fori_loop(0, n_per_w, one_roi, None)
    pltpu.make_async_copy(outb.at[0], out_hbm.at[pl.ds(0, _ROW)], semo1).wait()
    pltpu.make_async_copy(outb.at[1], out_hbm.at[pl.ds(0, _ROW)], semo0).wait()


def kernel(input, rois):
    N, C, H, W = input.shape
    nroi = rois.shape[0]
    x_hwc = jnp.transpose(input, (0, 2, 3, 1)).reshape(N * H, W, C)
    params = _make_params(rois, H, W)

    info = plsc.get_sparse_core_info()
    nc, ns = info.num_cores, info.num_subcores
    nw = nc * ns
    n_per_w = nroi // nw
    mesh = plsc.VectorSubcoreMesh(core_axis_name="c", subcore_axis_name="s")

    sc = pl.kernel(
        functools.partial(_roi_body, n_per_w=n_per_w, nc=nc),
        out_type=jax.ShapeDtypeStruct((nroi * _PP * _C,), jnp.float32),
        mesh=mesh,
        scratch_types=[
            pltpu.VMEM((n_per_w, 32), jnp.int16),
            pltpu.VMEM((2, _WIN, _WIN, _C), jnp.float32),
            pltpu.VMEM((2, _P * _C), jnp.float32),
            pltpu.SMEM((2 * _P,), jnp.int32),
            pltpu.SemaphoreType.DMA,
            pltpu.SemaphoreType.DMA,
            pltpu.SemaphoreType.DMA,
            pltpu.SemaphoreType.DMA,
        ],
        compiler_params=pltpu.CompilerParams(
            use_tc_tiling_on_sc=False, needs_layout_passes=False),
        name="roipool_sc",
    )
    out = sc(x_hwc, params)
    return out.reshape(nroi, _PP, _C).transpose(0, 2, 1).reshape(
        nroi, _C, _P, _P)
